# baseline (speedup 1.0000x reference)
import functools

import jax
import jax.numpy as jnp
from jax import lax
from jax.experimental import pallas as pl
from jax.experimental.pallas import tpu as pltpu
from jax.experimental.pallas import tpu_sc as plsc

EMB = 128
LANES = 16
NCORES = 1
NSUB = 16
NW = NCORES * NSUB
BATCH = 16384
BPW = BATCH // NW
CHUNK = 128
NCHUNK = BPW // CHUNK
GROUPS = CHUNK // LANES

_mesh = plsc.VectorSubcoreMesh(
    core_axis_name="c", subcore_axis_name="s", num_cores=NCORES)


@functools.partial(
    pl.kernel,
    out_type=jax.ShapeDtypeStruct((BATCH,), jnp.float32),
    mesh=_mesh,
    compiler_params=pltpu.CompilerParams(needs_layout_passes=False),
    scratch_types=[
        pltpu.VMEM((BPW,), jnp.int32),
        pltpu.VMEM((BPW,), jnp.int32),
        pltpu.VMEM((CHUNK, EMB), jnp.float32),
        pltpu.VMEM((CHUNK, EMB), jnp.float32),
        pltpu.VMEM((CHUNK, EMB), jnp.float32),
        pltpu.VMEM((CHUNK, EMB), jnp.float32),
        pltpu.VMEM((BPW,), jnp.float32),
        pltpu.VMEM((LANES,), jnp.float32),
        pltpu.SemaphoreType.DMA,
        pltpu.SemaphoreType.DMA,
    ],
)
def _bpr_sc(uids_hbm, iids_hbm, uemb_hbm, iemb_hbm, bias_hbm, out_hbm,
            uidx, iidx, urows0, urows1, irows0, irows1, outb, biasb,
            sem0, sem1):
    wid = lax.axis_index("s") * NCORES + lax.axis_index("c")
    base = wid * BPW
    cp_b = pltpu.async_copy(bias_hbm, biasb, sem0)
    cp_u = pltpu.async_copy(uids_hbm.at[pl.ds(base, BPW)], uidx, sem0)
    cp_i = pltpu.async_copy(iids_hbm.at[pl.ds(base, BPW)], iidx, sem0)
    cp_b.wait()
    cp_u.wait()
    cp_i.wait()
    bias_vec = biasb[...]
    row_iota = lax.iota(jnp.int32, LANES)

    bufs = [(urows0, irows0, sem0), (urows1, irows1, sem1)]

    def issue(c):
        ub, ib, sem = bufs[c & 1]
        return (
            pltpu.async_copy(
                uemb_hbm.at[uidx.at[pl.ds(c * CHUNK, CHUNK)]], ub, sem),
            pltpu.async_copy(
                iemb_hbm.at[iidx.at[pl.ds(c * CHUNK, CHUNK)]], ib, sem),
        )

    pending = issue(0)
    for c in range(NCHUNK):
        nxt = issue(c + 1) if c + 1 < NCHUNK else ()
        for cp in pending:
            cp.wait()
        pending = nxt
        urows, irows, _ = bufs[c & 1]

        def group_body(g, _, c=c, urows=urows, irows=irows):
            ov = bias_vec
            for r in range(LANES):
                row = g * LANES + r
                a = urows[row, pl.ds(0, LANES)] * irows[row, pl.ds(0, LANES)]
                for j in range(1, EMB // LANES):
                    a = a + (urows[row, pl.ds(j * LANES, LANES)]
                             * irows[row, pl.ds(j * LANES, LANES)])
                ov = jnp.where(row_iota == r, ov + jnp.sum(a), ov)
            outb[pl.ds(c * CHUNK + g * LANES, LANES)] = ov
            return 0

        lax.fori_loop(0, GROUPS, group_body, 0)

    pltpu.sync_copy(outb, out_hbm.at[pl.ds(base, BPW)])


def kernel(user_ids, item_ids, user_emb, item_emb, global_bias):
    bias_vec = jnp.full((LANES,), global_bias, dtype=jnp.float32)
    return _bpr_sc(user_ids, item_ids, user_emb, item_emb, bias_vec)


# speedup vs baseline: 1.2414x; 1.2414x over previous
import functools

import jax
import jax.numpy as jnp
from jax import lax
from jax.experimental import pallas as pl
from jax.experimental.pallas import tpu as pltpu
from jax.experimental.pallas import tpu_sc as plsc

EMB = 128
LANES = 16
NCORES = 2
NSUB = 16
NW = NCORES * NSUB
BATCH = 16384
BPW = BATCH // NW
CHUNK = 128
NCHUNK = BPW // CHUNK
GROUPS = CHUNK // LANES

_mesh = plsc.VectorSubcoreMesh(core_axis_name="c", subcore_axis_name="s")


@functools.partial(
    pl.kernel,
    out_type=jax.ShapeDtypeStruct((BATCH,), jnp.float32),
    mesh=_mesh,
    compiler_params=pltpu.CompilerParams(needs_layout_passes=False),
    scratch_types=[
        pltpu.VMEM((BPW,), jnp.int32),
        pltpu.VMEM((BPW,), jnp.int32),
        pltpu.VMEM((CHUNK, EMB), jnp.float32),
        pltpu.VMEM((CHUNK, EMB), jnp.float32),
        pltpu.VMEM((CHUNK, EMB), jnp.float32),
        pltpu.VMEM((CHUNK, EMB), jnp.float32),
        pltpu.VMEM((CHUNK, EMB), jnp.float32),
        pltpu.VMEM((CHUNK, EMB), jnp.float32),
        pltpu.VMEM((BPW,), jnp.float32),
        pltpu.VMEM((LANES,), jnp.float32),
        pltpu.SemaphoreType.DMA,
        pltpu.SemaphoreType.DMA,
        pltpu.SemaphoreType.DMA,
    ],
)
def _bpr_sc(uids_hbm, iids_hbm, uemb_hbm, iemb_hbm, bias_hbm, out_hbm,
            uidx, iidx, urows0, urows1, urows2, irows0, irows1, irows2,
            outb, biasb, sem0, sem1, sem2):
    wid = lax.axis_index("s") * NCORES + lax.axis_index("c")
    base = wid * BPW
    cp_b = pltpu.async_copy(bias_hbm, biasb, sem0)
    cp_u = pltpu.async_copy(uids_hbm.at[pl.ds(base, BPW)], uidx, sem0)
    cp_i = pltpu.async_copy(iids_hbm.at[pl.ds(base, BPW)], iidx, sem0)
    cp_b.wait()
    cp_u.wait()
    cp_i.wait()
    bias_vec = biasb[...]
    row_iota = lax.iota(jnp.int32, LANES)

    bufs = [(urows0, irows0, sem0), (urows1, irows1, sem1),
            (urows2, irows2, sem2)]
    DEPTH = len(bufs)

    def issue(c):
        ub, ib, sem = bufs[c % DEPTH]
        return (
            pltpu.async_copy(
                uemb_hbm.at[uidx.at[pl.ds(c * CHUNK, CHUNK)]], ub, sem),
            pltpu.async_copy(
                iemb_hbm.at[iidx.at[pl.ds(c * CHUNK, CHUNK)]], ib, sem),
        )

    pending = {c: issue(c) for c in range(min(DEPTH - 1, NCHUNK))}
    for c in range(NCHUNK):
        if c + DEPTH - 1 < NCHUNK:
            pending[c + DEPTH - 1] = issue(c + DEPTH - 1)
        for cp in pending.pop(c):
            cp.wait()
        urows, irows, _ = bufs[c % DEPTH]

        def group_body(g, _, c=c, urows=urows, irows=irows):
            ov = bias_vec
            for r in range(LANES):
                row = g * LANES + r
                a = urows[row, pl.ds(0, LANES)] * irows[row, pl.ds(0, LANES)]
                for j in range(1, EMB // LANES):
                    a = a + (urows[row, pl.ds(j * LANES, LANES)]
                             * irows[row, pl.ds(j * LANES, LANES)])
                ov = jnp.where(row_iota == r, ov + jnp.sum(a), ov)
            outb[pl.ds(c * CHUNK + g * LANES, LANES)] = ov
            return 0

        lax.fori_loop(0, GROUPS, group_body, 0)

    pltpu.sync_copy(outb, out_hbm.at[pl.ds(base, BPW)])


def kernel(user_ids, item_ids, user_emb, item_emb, global_bias):
    bias_vec = jnp.full((LANES,), global_bias, dtype=jnp.float32)
    return _bpr_sc(user_ids, item_ids, user_emb, item_emb, bias_vec)


# speedup vs baseline: 1.5253x; 1.2288x over previous
import functools

import jax
import jax.numpy as jnp
from jax import lax
from jax.experimental import pallas as pl
from jax.experimental.pallas import tpu as pltpu
from jax.experimental.pallas import tpu_sc as plsc

EMB = 128
LANES = 16
NCORES = 2
NSUB = 16
NW = NCORES * NSUB
BATCH = 16384
BPW = BATCH // NW
CHUNK = 128
NCHUNK = BPW // CHUNK
GROUPS = CHUNK // LANES

_mesh = plsc.VectorSubcoreMesh(core_axis_name="c", subcore_axis_name="s")


@functools.partial(
    pl.kernel,
    out_type=jax.ShapeDtypeStruct((BATCH,), jnp.float32),
    mesh=_mesh,
    compiler_params=pltpu.CompilerParams(needs_layout_passes=False),
    scratch_types=[
        pltpu.VMEM((BPW,), jnp.int32),
        pltpu.VMEM((BPW,), jnp.int32),
        pltpu.VMEM((CHUNK, EMB), jnp.float32),
        pltpu.VMEM((CHUNK, EMB), jnp.float32),
        pltpu.VMEM((CHUNK, EMB), jnp.float32),
        pltpu.VMEM((CHUNK, EMB), jnp.float32),
        pltpu.VMEM((BPW,), jnp.float32),
        pltpu.VMEM((LANES,), jnp.float32),
        pltpu.SemaphoreType.DMA,
        pltpu.SemaphoreType.DMA,
    ],
)
def _bpr_sc(uids_hbm, iids_hbm, uemb_hbm, iemb_hbm, bias_hbm, out_hbm,
            uidx, iidx, urows0, urows1, irows0, irows1, outb, biasb,
            sem0, sem1):
    wid = lax.axis_index("s") * NCORES + lax.axis_index("c")
    base = wid * BPW
    cp_b = pltpu.async_copy(bias_hbm, biasb, sem0)
    cp_u = pltpu.async_copy(uids_hbm.at[pl.ds(base, BPW)], uidx, sem0)
    cp_i = pltpu.async_copy(iids_hbm.at[pl.ds(base, BPW)], iidx, sem0)
    cp_b.wait()
    cp_u.wait()
    cp_i.wait()
    bias_vec = biasb[...]
    row_iota = lax.iota(jnp.int32, LANES)

    bufs = [(urows0, irows0, sem0), (urows1, irows1, sem1)]

    def issue(c):
        ub, ib, sem = bufs[c & 1]
        return (
            pltpu.async_copy(
                uemb_hbm.at[uidx.at[pl.ds(c * CHUNK, CHUNK)]], ub, sem),
            pltpu.async_copy(
                iemb_hbm.at[iidx.at[pl.ds(c * CHUNK, CHUNK)]], ib, sem),
        )

    pending = issue(0)
    for c in range(NCHUNK):
        nxt = issue(c + 1) if c + 1 < NCHUNK else ()
        for cp in pending:
            cp.wait()
        pending = nxt
        urows, irows, _ = bufs[c & 1]

        def group_body(g, _, c=c, urows=urows, irows=irows):
            def quad_body(q, ov, g=g, urows=urows, irows=irows):
                for k in range(4):
                    row = g * LANES + q * 4 + k
                    a = (urows[row, pl.ds(0, LANES)]
                         * irows[row, pl.ds(0, LANES)])
                    for j in range(1, EMB // LANES):
                        a = a + (urows[row, pl.ds(j * LANES, LANES)]
                                 * irows[row, pl.ds(j * LANES, LANES)])
                    ov = jnp.where(row_iota == q * 4 + k, ov + jnp.sum(a), ov)
                return ov

            ov = lax.fori_loop(0, 4, quad_body, bias_vec)
            outb[pl.ds(c * CHUNK + g * LANES, LANES)] = ov
            return 0

        lax.fori_loop(0, GROUPS, group_body, 0)

    pltpu.sync_copy(outb, out_hbm.at[pl.ds(base, BPW)])


def kernel(user_ids, item_ids, user_emb, item_emb, global_bias):
    bias_vec = jnp.full((LANES,), global_bias, dtype=jnp.float32)
    return _bpr_sc(user_ids, item_ids, user_emb, item_emb, bias_vec)


# speedup vs baseline: 1.5668x; 1.0272x over previous
import functools

import jax
import jax.numpy as jnp
from jax import lax
from jax.experimental import pallas as pl
from jax.experimental.pallas import tpu as pltpu
from jax.experimental.pallas import tpu_sc as plsc

EMB = 128
LANES = 16
NCORES = 2
NSUB = 16
NW = NCORES * NSUB
BATCH = 16384
BPW = BATCH // NW
CHUNK = 128
NCHUNK = BPW // CHUNK
NPAIR = NCHUNK // 2
GROUPS = CHUNK // LANES

_mesh = plsc.VectorSubcoreMesh(core_axis_name="c", subcore_axis_name="s")


@functools.partial(
    pl.kernel,
    out_type=jax.ShapeDtypeStruct((BATCH,), jnp.float32),
    mesh=_mesh,
    compiler_params=pltpu.CompilerParams(needs_layout_passes=False),
    scratch_types=[
        pltpu.VMEM((BPW,), jnp.int32),
        pltpu.VMEM((BPW,), jnp.int32),
        pltpu.VMEM((CHUNK, EMB), jnp.float32),
        pltpu.VMEM((CHUNK, EMB), jnp.float32),
        pltpu.VMEM((CHUNK, EMB), jnp.float32),
        pltpu.VMEM((CHUNK, EMB), jnp.float32),
        pltpu.VMEM((BPW,), jnp.float32),
        pltpu.VMEM((LANES,), jnp.float32),
        pltpu.SemaphoreType.DMA,
        pltpu.SemaphoreType.DMA,
    ],
)
def _bpr_sc(uids_hbm, iids_hbm, uemb_hbm, iemb_hbm, bias_hbm, out_hbm,
            uidx, iidx, urows0, urows1, irows0, irows1, outb, biasb,
            sem0, sem1):
    wid = lax.axis_index("s") * NCORES + lax.axis_index("c")
    base = wid * BPW
    cp_b = pltpu.async_copy(bias_hbm, biasb, sem0)
    cp_u = pltpu.async_copy(uids_hbm.at[pl.ds(base, BPW)], uidx, sem0)
    cp_i = pltpu.async_copy(iids_hbm.at[pl.ds(base, BPW)], iidx, sem0)
    cp_b.wait()
    cp_u.wait()
    cp_i.wait()
    bias_vec = biasb[...]
    row_iota = lax.iota(jnp.int32, LANES)

    bufs = [(urows0, irows0, sem0), (urows1, irows1, sem1)]

    def gathers(c, p):
        ub, ib, sem = bufs[p]
        return (
            pltpu.make_async_copy(
                uemb_hbm.at[uidx.at[pl.ds(c * CHUNK, CHUNK)]], ub, sem),
            pltpu.make_async_copy(
                iemb_hbm.at[iidx.at[pl.ds(c * CHUNK, CHUNK)]], ib, sem),
        )

    for cp in gathers(0, 0) + gathers(1, 1):
        cp.start()

    def pair_body(t, _):
        for p in range(2):
            c = 2 * t + p
            urows, irows, _ = bufs[p]
            for cp in gathers(c, p):
                cp.wait()

            def group_body(g, _, urows=urows, irows=irows, c=c):
                def quad_body(q, ov, g=g, urows=urows, irows=irows):
                    for k in range(4):
                        row = g * LANES + q * 4 + k
                        a = (urows[row, pl.ds(0, LANES)]
                             * irows[row, pl.ds(0, LANES)])
                        for j in range(1, EMB // LANES):
                            a = a + (urows[row, pl.ds(j * LANES, LANES)]
                                     * irows[row, pl.ds(j * LANES, LANES)])
                        ov = jnp.where(row_iota == q * 4 + k,
                                       ov + jnp.sum(a), ov)
                    return ov

                ov = lax.fori_loop(0, 4, quad_body, bias_vec)
                outb[pl.ds(c * CHUNK + g * LANES, LANES)] = ov
                return 0

            lax.fori_loop(0, GROUPS, group_body, 0)

            @pl.when(t < NPAIR - 1)
            def _():
                for cp in gathers(c + 2, p):
                    cp.start()

        return 0

    lax.fori_loop(0, NPAIR, pair_body, 0)
    pltpu.sync_copy(outb, out_hbm.at[pl.ds(base, BPW)])


def kernel(user_ids, item_ids, user_emb, item_emb, global_bias):
    bias_vec = jnp.full((LANES,), global_bias, dtype=jnp.float32)
    return _bpr_sc(user_ids, item_ids, user_emb, item_emb, bias_vec)


# speedup vs baseline: 1.5894x; 1.0144x over previous
import functools

import jax
import jax.numpy as jnp
from jax import lax
from jax.experimental import pallas as pl
from jax.experimental.pallas import tpu as pltpu
from jax.experimental.pallas import tpu_sc as plsc

EMB = 128
LANES = 16
NCORES = 2
NSUB = 16
NW = NCORES * NSUB
BATCH = 16384
BPW = BATCH // NW
CHUNK = 64
NCHUNK = BPW // CHUNK
NPAIR = NCHUNK // 2
GROUPS = CHUNK // LANES

_mesh = plsc.VectorSubcoreMesh(core_axis_name="c", subcore_axis_name="s")


@functools.partial(
    pl.kernel,
    out_type=jax.ShapeDtypeStruct((BATCH,), jnp.float32),
    mesh=_mesh,
    compiler_params=pltpu.CompilerParams(needs_layout_passes=False),
    scratch_types=[
        pltpu.VMEM((BPW,), jnp.int32),
        pltpu.VMEM((BPW,), jnp.int32),
        pltpu.VMEM((CHUNK, EMB), jnp.float32),
        pltpu.VMEM((CHUNK, EMB), jnp.float32),
        pltpu.VMEM((CHUNK, EMB), jnp.float32),
        pltpu.VMEM((CHUNK, EMB), jnp.float32),
        pltpu.VMEM((BPW,), jnp.float32),
        pltpu.VMEM((LANES,), jnp.float32),
        pltpu.SemaphoreType.DMA,
        pltpu.SemaphoreType.DMA,
    ],
)
def _bpr_sc(uids_hbm, iids_hbm, uemb_hbm, iemb_hbm, bias_hbm, out_hbm,
            uidx, iidx, urows0, urows1, irows0, irows1, outb, biasb,
            sem0, sem1):
    wid = lax.axis_index("s") * NCORES + lax.axis_index("c")
    base = wid * BPW
    cp_b = pltpu.async_copy(bias_hbm, biasb, sem0)
    cp_u = pltpu.async_copy(uids_hbm.at[pl.ds(base, BPW)], uidx, sem0)
    cp_i = pltpu.async_copy(iids_hbm.at[pl.ds(base, BPW)], iidx, sem0)
    cp_b.wait()
    cp_u.wait()
    cp_i.wait()
    bias_vec = biasb[...]
    row_iota = lax.iota(jnp.int32, LANES)

    bufs = [(urows0, irows0, sem0), (urows1, irows1, sem1)]

    def gathers(c, p):
        ub, ib, sem = bufs[p]
        return (
            pltpu.make_async_copy(
                uemb_hbm.at[uidx.at[pl.ds(c * CHUNK, CHUNK)]], ub, sem),
            pltpu.make_async_copy(
                iemb_hbm.at[iidx.at[pl.ds(c * CHUNK, CHUNK)]], ib, sem),
        )

    for cp in gathers(0, 0) + gathers(1, 1):
        cp.start()

    def pair_body(t, _):
        for p in range(2):
            c = 2 * t + p
            urows, irows, _ = bufs[p]
            for cp in gathers(c, p):
                cp.wait()

            def group_body(g, _, urows=urows, irows=irows, c=c):
                def quad_body(q, ov, g=g, urows=urows, irows=irows):
                    for k in range(4):
                        row = g * LANES + q * 4 + k
                        a = (urows[row, pl.ds(0, LANES)]
                             * irows[row, pl.ds(0, LANES)])
                        for j in range(1, EMB // LANES):
                            a = a + (urows[row, pl.ds(j * LANES, LANES)]
                                     * irows[row, pl.ds(j * LANES, LANES)])
                        ov = jnp.where(row_iota == q * 4 + k,
                                       ov + jnp.sum(a), ov)
                    return ov

                ov = lax.fori_loop(0, 4, quad_body, bias_vec)
                outb[pl.ds(c * CHUNK + g * LANES, LANES)] = ov
                return 0

            lax.fori_loop(0, GROUPS, group_body, 0)

            @pl.when(t < NPAIR - 1)
            def _():
                for cp in gathers(c + 2, p):
                    cp.start()

        return 0

    lax.fori_loop(0, NPAIR, pair_body, 0)
    pltpu.sync_copy(outb, out_hbm.at[pl.ds(base, BPW)])


def kernel(user_ids, item_ids, user_emb, item_emb, global_bias):
    bias_vec = jnp.full((LANES,), global_bias, dtype=jnp.float32)
    return _bpr_sc(user_ids, item_ids, user_emb, item_emb, bias_vec)
